# Initial kernel scaffold; baseline (speedup 1.0000x reference)
import jax
import jax.numpy as jnp
from jax.experimental import pallas as pl


def kernel(features, edge_index, W1, b1, W2, b2):
    raise NotImplementedError("write your pallas kernel here")


# speedup vs baseline: 1.1867x; 1.1867x over previous
import functools

import jax
import jax.numpy as jnp
from jax import lax
from jax.experimental import pallas as pl
from jax.experimental.pallas import tpu as pltpu
from jax.experimental.pallas import tpu_sc as plsc

N_NODES_K = 10000
D_K = 128
E_K = 320000

NC = 2
NS = 16
NW = NC * NS
CHUNK = 128
CPW = 80
E_PAD = NW * CPW * CHUNK
N_PAD = 10240
ROWS_PW = CPW * CHUNK

_MESH = plsc.VectorSubcoreMesh(core_axis_name="c", subcore_axis_name="s")


@functools.partial(
    pl.kernel,
    out_type=jax.ShapeDtypeStruct((E_PAD, D_K), jnp.float32),
    mesh=_MESH,
    scratch_types=[
        pltpu.VMEM((CPW, CHUNK), jnp.int32),
        pltpu.VMEM((CHUNK, D_K), jnp.float32),
        pltpu.VMEM((CHUNK, D_K), jnp.float32),
        pltpu.VMEM((16,), jnp.int32),
        pltpu.VMEM((16, D_K), jnp.float32),
    ],
)
def _gather_kernel(h_hbm, src_hbm, out_hbm, idx, r0, r1, fidx, fbuf):
    c = lax.axis_index("c")
    s = lax.axis_index("s")
    w = c * NS + s
    iota16 = lax.iota(jnp.int32, 16)
    fidx[...] = iota16

    pltpu.sync_copy(src_hbm.at[pl.ds(w * CPW, CPW)], idx)

    obase = w * ROWS_PW
    bufs = [r0, r1]
    pltpu.sync_copy(h_hbm.at[idx.at[0]], bufs[0])
    for kk in range(CPW):
        if kk + 1 < CPW:
            pltpu.sync_copy(h_hbm.at[idx.at[kk + 1]], bufs[(kk + 1) % 2])
        else:
            pltpu.sync_copy(h_hbm.at[fidx], fbuf)
        pltpu.sync_copy(bufs[kk % 2],
                        out_hbm.at[pl.ds(obase + kk * CHUNK, CHUNK)])


_BLK = 256


def _prescale(x_pad, deg_out):
    def body(x_ref, d_ref, o_ref):
        deg = jnp.clip(d_ref[...], 1.0, None)
        o_ref[...] = x_ref[...] * lax.rsqrt(deg)

    return pl.pallas_call(
        body,
        grid=(N_PAD // _BLK,),
        in_specs=[
            pl.BlockSpec((_BLK, D_K), lambda i: (i, 0)),
            pl.BlockSpec((_BLK, 1), lambda i: (i, 0)),
        ],
        out_specs=pl.BlockSpec((_BLK, D_K), lambda i: (i, 0)),
        out_shape=jax.ShapeDtypeStruct((N_PAD, D_K), jnp.float32),
    )(x_pad, deg_out)


def _affine(agg, deg_in, W, b, deg_out=None):
    relu_scale = deg_out is not None

    def body(a_ref, di_ref, w_ref, b_ref, *rest):
        if relu_scale:
            do_ref, o_ref = rest
        else:
            (o_ref,) = rest
        di = jnp.clip(di_ref[...], 1.0, None)
        y = jnp.dot(a_ref[...] * lax.rsqrt(di), w_ref[...],
                    preferred_element_type=jnp.float32) + b_ref[...]
        if relu_scale:
            do = jnp.clip(do_ref[...], 1.0, None)
            y = jnp.maximum(y, 0.0) * lax.rsqrt(do)
        o_ref[...] = y

    in_specs = [
        pl.BlockSpec((_BLK, D_K), lambda i: (i, 0)),
        pl.BlockSpec((_BLK, 1), lambda i: (i, 0)),
        pl.BlockSpec((D_K, D_K), lambda i: (0, 0)),
        pl.BlockSpec((1, D_K), lambda i: (0, 0)),
    ]
    args = [agg, deg_in, W, b.reshape(1, D_K)]
    if relu_scale:
        in_specs.append(pl.BlockSpec((_BLK, 1), lambda i: (i, 0)))
        args.append(deg_out)
    return pl.pallas_call(
        body,
        grid=(N_PAD // _BLK,),
        in_specs=in_specs,
        out_specs=pl.BlockSpec((_BLK, D_K), lambda i: (i, 0)),
        out_shape=jax.ShapeDtypeStruct((N_PAD, D_K), jnp.float32),
    )(*args)


def kernel(features, edge_index, W1, b1, W2, b2):
    src = edge_index[0].astype(jnp.int32)
    dst = edge_index[1].astype(jnp.int32)
    padv = jnp.full((E_PAD - E_K,), N_NODES_K, jnp.int32)
    srcp = jnp.concatenate([src, padv]).reshape(E_PAD // CHUNK, CHUNK)
    dst_pad = jnp.concatenate([dst, padv])
    xp = jnp.pad(features, ((0, N_PAD - N_NODES_K), (0, 0)))

    deg_out = jnp.bincount(src, length=N_PAD).astype(jnp.float32).reshape(-1, 1)
    deg_in = jnp.bincount(dst, length=N_PAD).astype(jnp.float32).reshape(-1, 1)

    h1 = _prescale(xp, deg_out)
    m1 = _gather_kernel(h1, srcp)
    a1 = jax.ops.segment_sum(m1, dst_pad, num_segments=N_PAD)
    h2 = _affine(a1, deg_in, W1, b1, deg_out=deg_out)
    m2 = _gather_kernel(h2, srcp)
    a2 = jax.ops.segment_sum(m2, dst_pad, num_segments=N_PAD)
    y = _affine(a2, deg_in, W2, b2)
    return y[:N_NODES_K]
